# baseline (speedup 1.0000x reference)
import functools

import jax
import jax.numpy as jnp
from jax import lax
from jax.experimental import pallas as pl
from jax.experimental.pallas import tpu as pltpu


def _pick_tile(b, target=128):
    best = 8
    for t in range(8, min(target, b) + 1, 8):
        if b % t == 0:
            best = t
    return best


def _body(x_ref, x2_ref, wh_ref, bh_ref, wp_ref, bp_ref,
          o1a_ref, o2a_ref, o1b_ref, o2b_ref, *, feat_dim, tb):
    xb = jnp.concatenate([x_ref[...], x2_ref[...]], axis=0).astype(jnp.bfloat16)
    f = jnp.dot(xb, wh_ref[...], preferred_element_type=jnp.float32)
    f = jnp.maximum(f + bh_ref[...], 0.0).astype(jnp.bfloat16)
    y = jnp.dot(f, wp_ref[...], preferred_element_type=jnp.float32) + bp_ref[...]
    y1 = y[:, :feat_dim]
    y2 = y[:, feat_dim:]
    n1 = y1 * lax.rsqrt(jnp.sum(y1 * y1, axis=-1, keepdims=True))
    n2 = y2 * lax.rsqrt(jnp.sum(y2 * y2, axis=-1, keepdims=True))
    o1a_ref[...] = n1[:tb].astype(o1a_ref.dtype)
    o2a_ref[...] = n2[:tb].astype(o2a_ref.dtype)
    o1b_ref[...] = n1[tb:].astype(o1b_ref.dtype)
    o2b_ref[...] = n2[tb:].astype(o2b_ref.dtype)


@jax.jit
def _run(x, x2, w_head, b_head, w_proj, b_proj):
    B, D = x.shape
    F2 = w_proj.shape[1]
    feat_dim = F2 // 2
    tb = _pick_tile(B)
    wh = w_head.astype(jnp.bfloat16)
    wp = w_proj.astype(jnp.bfloat16)
    out_block = pl.BlockSpec((tb, feat_dim), lambda i: (i, 0))
    return pl.pallas_call(
        functools.partial(_body, feat_dim=feat_dim, tb=tb),
        out_shape=tuple(jax.ShapeDtypeStruct((B, feat_dim), x.dtype)
                        for _ in range(4)),
        grid=(B // tb,),
        in_specs=[
            pl.BlockSpec((tb, D), lambda i: (i, 0)),
            pl.BlockSpec((tb, D), lambda i: (i, 0)),
            pl.BlockSpec((D, D), lambda i: (0, 0)),
            pl.BlockSpec((1, D), lambda i: (0, 0)),
            pl.BlockSpec((D, F2), lambda i: (0, 0)),
            pl.BlockSpec((1, F2), lambda i: (0, 0)),
        ],
        out_specs=(out_block, out_block, out_block, out_block),
        compiler_params=pltpu.CompilerParams(
            dimension_semantics=("parallel",),
            vmem_limit_bytes=64 * 1024 * 1024,
        ),
    )(x, x2, wh, b_head, wp, b_proj)


def kernel(x, x2, w_head, b_head, w_proj, b_proj):
    return _run(x, x2, w_head, b_head, w_proj, b_proj)


# speedup vs baseline: 1.0884x; 1.0884x over previous
import functools

import jax
import jax.numpy as jnp
from jax import lax
from jax.experimental import pallas as pl
from jax.experimental.pallas import tpu as pltpu


def _pick_tile(b, target=512):
    best = 8
    for t in range(8, min(target, b) + 1, 8):
        if b % t == 0:
            best = t
    return best


def _body(x_ref, x2_ref, wh_ref, bh_ref, wp_ref, bp_ref,
          o1a_ref, o2a_ref, o1b_ref, o2b_ref, *, feat_dim, tb):
    def _one(xv, o1_ref, o2_ref):
        f = jnp.dot(xv.astype(jnp.bfloat16), wh_ref[...],
                    preferred_element_type=jnp.float32)
        f = jnp.maximum(f + bh_ref[...], 0.0).astype(jnp.bfloat16)
        y = jnp.dot(f, wp_ref[...], preferred_element_type=jnp.float32) + bp_ref[...]
        y1 = y[:, :feat_dim]
        y2 = y[:, feat_dim:]
        o1_ref[...] = (y1 * lax.rsqrt(jnp.sum(y1 * y1, axis=-1, keepdims=True))
                       ).astype(o1_ref.dtype)
        o2_ref[...] = (y2 * lax.rsqrt(jnp.sum(y2 * y2, axis=-1, keepdims=True))
                       ).astype(o2_ref.dtype)

    _one(x_ref[...], o1a_ref, o2a_ref)
    _one(x2_ref[...], o1b_ref, o2b_ref)


@jax.jit
def _run(x, x2, w_head, b_head, w_proj, b_proj):
    B, D = x.shape
    F2 = w_proj.shape[1]
    feat_dim = F2 // 2
    tb = _pick_tile(B)
    wh = w_head.astype(jnp.bfloat16)
    wp = w_proj.astype(jnp.bfloat16)
    out_block = pl.BlockSpec((tb, feat_dim), lambda i: (i, 0))
    return pl.pallas_call(
        functools.partial(_body, feat_dim=feat_dim, tb=tb),
        out_shape=tuple(jax.ShapeDtypeStruct((B, feat_dim), x.dtype)
                        for _ in range(4)),
        grid=(B // tb,),
        in_specs=[
            pl.BlockSpec((tb, D), lambda i: (i, 0)),
            pl.BlockSpec((tb, D), lambda i: (i, 0)),
            pl.BlockSpec((D, D), lambda i: (0, 0)),
            pl.BlockSpec((1, D), lambda i: (0, 0)),
            pl.BlockSpec((D, F2), lambda i: (0, 0)),
            pl.BlockSpec((1, F2), lambda i: (0, 0)),
        ],
        out_specs=(out_block, out_block, out_block, out_block),
        compiler_params=pltpu.CompilerParams(
            dimension_semantics=("parallel",),
            vmem_limit_bytes=64 * 1024 * 1024,
        ),
    )(x, x2, wh, b_head, wp, b_proj)


def kernel(x, x2, w_head, b_head, w_proj, b_proj):
    return _run(x, x2, w_head, b_head, w_proj, b_proj)


# speedup vs baseline: 1.1602x; 1.0659x over previous
import functools

import jax
import jax.numpy as jnp
from jax import lax
from jax.experimental import pallas as pl
from jax.experimental.pallas import tpu as pltpu


def _pick_tile(b, target=256):
    best = 8
    for t in range(8, min(target, b) + 1, 8):
        if b % t == 0:
            best = t
    return best


def _body(x_ref, x2_ref, wh_ref, bh_ref, wp_ref, bp_ref,
          o1a_ref, o2a_ref, o1b_ref, o2b_ref, whb_ref, wpb_ref, *, feat_dim):
    k = pl.program_id(1)

    @pl.when(k == 0)
    def _():
        whb_ref[...] = wh_ref[...].astype(jnp.bfloat16)
        wpb_ref[...] = wp_ref[...].astype(jnp.bfloat16)

    def _one(xv, o1_ref, o2_ref):
        f = jnp.dot(xv.astype(jnp.bfloat16), whb_ref[...],
                    preferred_element_type=jnp.float32)
        f = jnp.maximum(f + bh_ref[...], 0.0).astype(jnp.bfloat16)
        y = jnp.dot(f, wpb_ref[...], preferred_element_type=jnp.float32) + bp_ref[...]
        y1 = y[:, :feat_dim]
        y2 = y[:, feat_dim:]
        o1_ref[...] = (y1 * lax.rsqrt(jnp.sum(y1 * y1, axis=-1, keepdims=True))
                       ).astype(o1_ref.dtype)
        o2_ref[...] = (y2 * lax.rsqrt(jnp.sum(y2 * y2, axis=-1, keepdims=True))
                       ).astype(o2_ref.dtype)

    _one(x_ref[...], o1a_ref, o2a_ref)
    _one(x2_ref[...], o1b_ref, o2b_ref)


@jax.jit
def _run(x, x2, w_head, b_head, w_proj, b_proj):
    B, D = x.shape
    F2 = w_proj.shape[1]
    feat_dim = F2 // 2
    half = B // 2
    tb = _pick_tile(half)
    steps = half // tb

    def _xmap(i, k):
        return (i * steps + k, 0)

    _const = lambda i, k: (0, 0)
    out_block = pl.BlockSpec((tb, feat_dim), _xmap)
    return pl.pallas_call(
        functools.partial(_body, feat_dim=feat_dim),
        out_shape=tuple(jax.ShapeDtypeStruct((B, feat_dim), x.dtype)
                        for _ in range(4)),
        grid=(2, steps),
        in_specs=[
            pl.BlockSpec((tb, D), _xmap),
            pl.BlockSpec((tb, D), _xmap),
            pl.BlockSpec((D, D), _const),
            pl.BlockSpec((1, D), _const),
            pl.BlockSpec((D, F2), _const),
            pl.BlockSpec((1, F2), _const),
        ],
        out_specs=(out_block, out_block, out_block, out_block),
        scratch_shapes=[
            pltpu.VMEM((D, D), jnp.bfloat16),
            pltpu.VMEM((D, F2), jnp.bfloat16),
        ],
        compiler_params=pltpu.CompilerParams(
            dimension_semantics=("parallel", "arbitrary"),
            vmem_limit_bytes=100 * 1024 * 1024,
        ),
    )(x, x2, w_head, b_head, w_proj, b_proj)


def kernel(x, x2, w_head, b_head, w_proj, b_proj):
    return _run(x, x2, w_head, b_head, w_proj, b_proj)


# speedup vs baseline: 1.1866x; 1.0228x over previous
import functools

import jax
import jax.numpy as jnp
from jax import lax
from jax.experimental import pallas as pl
from jax.experimental.pallas import tpu as pltpu


def _pick_tile(b, target=512):
    best = 8
    for t in range(8, min(target, b) + 1, 8):
        if b % t == 0:
            best = t
    return best


def _body(x_ref, x2_ref, wh_ref, bh_ref, wp_ref, bp_ref,
          o1a_ref, o2a_ref, o1b_ref, o2b_ref, whb_ref, wpb_ref, *, feat_dim):
    k = pl.program_id(1)

    @pl.when(k == 0)
    def _():
        whb_ref[...] = wh_ref[...].astype(jnp.bfloat16)
        wpb_ref[...] = wp_ref[...].astype(jnp.bfloat16)

    def _one(xv, o1_ref, o2_ref):
        f = jnp.dot(xv.astype(jnp.bfloat16), whb_ref[...],
                    preferred_element_type=jnp.float32)
        f = jnp.maximum(f + bh_ref[...], 0.0).astype(jnp.bfloat16)
        y = jnp.dot(f, wpb_ref[...], preferred_element_type=jnp.float32) + bp_ref[...]
        y1 = y[:, :feat_dim]
        y2 = y[:, feat_dim:]
        o1_ref[...] = (y1 * lax.rsqrt(jnp.sum(y1 * y1, axis=-1, keepdims=True))
                       ).astype(o1_ref.dtype)
        o2_ref[...] = (y2 * lax.rsqrt(jnp.sum(y2 * y2, axis=-1, keepdims=True))
                       ).astype(o2_ref.dtype)

    _one(x_ref[...], o1a_ref, o2a_ref)
    _one(x2_ref[...], o1b_ref, o2b_ref)


@jax.jit
def _run(x, x2, w_head, b_head, w_proj, b_proj):
    B, D = x.shape
    F2 = w_proj.shape[1]
    feat_dim = F2 // 2
    half = B // 2
    tb = _pick_tile(half)
    steps = half // tb

    def _xmap(i, k):
        return (i * steps + k, 0)

    _const = lambda i, k: (0, 0)
    out_block = pl.BlockSpec((tb, feat_dim), _xmap)
    return pl.pallas_call(
        functools.partial(_body, feat_dim=feat_dim),
        out_shape=tuple(jax.ShapeDtypeStruct((B, feat_dim), x.dtype)
                        for _ in range(4)),
        grid=(2, steps),
        in_specs=[
            pl.BlockSpec((tb, D), _xmap),
            pl.BlockSpec((tb, D), _xmap),
            pl.BlockSpec((D, D), _const),
            pl.BlockSpec((1, D), _const),
            pl.BlockSpec((D, F2), _const),
            pl.BlockSpec((1, F2), _const),
        ],
        out_specs=(out_block, out_block, out_block, out_block),
        scratch_shapes=[
            pltpu.VMEM((D, D), jnp.bfloat16),
            pltpu.VMEM((D, F2), jnp.bfloat16),
        ],
        compiler_params=pltpu.CompilerParams(
            dimension_semantics=("parallel", "arbitrary"),
            vmem_limit_bytes=100 * 1024 * 1024,
        ),
    )(x, x2, w_head, b_head, w_proj, b_proj)


def kernel(x, x2, w_head, b_head, w_proj, b_proj):
    return _run(x, x2, w_head, b_head, w_proj, b_proj)


# speedup vs baseline: 1.1883x; 1.0015x over previous
import functools

import jax
import jax.numpy as jnp
from jax import lax
from jax.experimental import pallas as pl
from jax.experimental.pallas import tpu as pltpu


def _pick_tile(b, target=512):
    best = 8
    for t in range(8, min(target, b) + 1, 8):
        if b % t == 0:
            best = t
    return best


def _body(x_ref, x2_ref, wh_ref, bh_ref, wp_ref, bp_ref,
          o1a_ref, o2a_ref, o1b_ref, o2b_ref, whb_ref, wpb_ref, *, feat_dim):
    k = pl.program_id(1)

    @pl.when(k == 0)
    def _():
        whb_ref[...] = wh_ref[...].astype(jnp.bfloat16)
        wpb_ref[...] = wp_ref[...].astype(jnp.bfloat16)

    def _one(xv, o1_ref, o2_ref):
        f = jnp.dot(xv.astype(jnp.bfloat16), whb_ref[...],
                    preferred_element_type=jnp.float32)
        f = jnp.maximum(f + bh_ref[...], 0.0).astype(jnp.bfloat16)
        y = jnp.dot(f, wpb_ref[...], preferred_element_type=jnp.float32) + bp_ref[...]
        y1 = y[:, :feat_dim]
        y2 = y[:, feat_dim:]
        o1_ref[...] = (y1 * lax.rsqrt(jnp.sum(y1 * y1, axis=-1, keepdims=True))
                       ).astype(o1_ref.dtype)
        o2_ref[...] = (y2 * lax.rsqrt(jnp.sum(y2 * y2, axis=-1, keepdims=True))
                       ).astype(o2_ref.dtype)

    tb = x_ref.shape[0]
    h = tb // 2
    _one(x_ref[:h], o1a_ref.at[:h], o2a_ref.at[:h])
    _one(x2_ref[:h], o1b_ref.at[:h], o2b_ref.at[:h])
    _one(x_ref[h:], o1a_ref.at[h:], o2a_ref.at[h:])
    _one(x2_ref[h:], o1b_ref.at[h:], o2b_ref.at[h:])


@jax.jit
def _run(x, x2, w_head, b_head, w_proj, b_proj):
    B, D = x.shape
    F2 = w_proj.shape[1]
    feat_dim = F2 // 2
    half = B // 2
    tb = _pick_tile(half)
    steps = half // tb

    def _xmap(i, k):
        return (i * steps + k, 0)

    _const = lambda i, k: (0, 0)
    out_block = pl.BlockSpec((tb, feat_dim), _xmap)
    return pl.pallas_call(
        functools.partial(_body, feat_dim=feat_dim),
        out_shape=tuple(jax.ShapeDtypeStruct((B, feat_dim), x.dtype)
                        for _ in range(4)),
        grid=(2, steps),
        in_specs=[
            pl.BlockSpec((tb, D), _xmap),
            pl.BlockSpec((tb, D), _xmap),
            pl.BlockSpec((D, D), _const),
            pl.BlockSpec((1, D), _const),
            pl.BlockSpec((D, F2), _const),
            pl.BlockSpec((1, F2), _const),
        ],
        out_specs=(out_block, out_block, out_block, out_block),
        scratch_shapes=[
            pltpu.VMEM((D, D), jnp.bfloat16),
            pltpu.VMEM((D, F2), jnp.bfloat16),
        ],
        compiler_params=pltpu.CompilerParams(
            dimension_semantics=("parallel", "arbitrary"),
            vmem_limit_bytes=100 * 1024 * 1024,
        ),
    )(x, x2, w_head, b_head, w_proj, b_proj)


def kernel(x, x2, w_head, b_head, w_proj, b_proj):
    return _run(x, x2, w_head, b_head, w_proj, b_proj)


# speedup vs baseline: 1.1904x; 1.0018x over previous
import functools

import jax
import jax.numpy as jnp
from jax import lax
from jax.experimental import pallas as pl
from jax.experimental.pallas import tpu as pltpu


def _pick_tile(b, target=512):
    best = 8
    for t in range(8, min(target, b) + 1, 8):
        if b % t == 0:
            best = t
    return best


def _outer_body(x_hbm, x2_hbm, wh_ref, bh_ref, wp_ref, bp_ref,
                o1a, o2a, o1b, o2b, whb_ref, wpb_ref, *, feat_dim, tb, steps):
    whb_ref[...] = wh_ref[...].astype(jnp.bfloat16)
    wpb_ref[...] = wp_ref[...].astype(jnp.bfloat16)

    def _one(xv, o1_ref, o2_ref):
        f = jnp.dot(xv.astype(jnp.bfloat16), whb_ref[...],
                    preferred_element_type=jnp.float32)
        f = jnp.maximum(f + bh_ref[...], 0.0).astype(jnp.bfloat16)
        y = jnp.dot(f, wpb_ref[...], preferred_element_type=jnp.float32) + bp_ref[...]
        y1 = y[:, :feat_dim]
        y2 = y[:, feat_dim:]
        o1_ref[...] = (y1 * lax.rsqrt(jnp.sum(y1 * y1, axis=-1, keepdims=True))
                       ).astype(o1_ref.dtype)
        o2_ref[...] = (y2 * lax.rsqrt(jnp.sum(y2 * y2, axis=-1, keepdims=True))
                       ).astype(o2_ref.dtype)

    def _step(x_ref, x2_ref, o1a_ref, o2a_ref, o1b_ref, o2b_ref):
        _one(x_ref[...], o1a_ref, o2a_ref)
        _one(x2_ref[...], o1b_ref, o2b_ref)

    D = wh_ref.shape[0]
    pipe = pltpu.emit_pipeline(
        _step,
        grid=(steps,),
        in_specs=[
            pl.BlockSpec((tb, D), lambda i: (i, 0)),
            pl.BlockSpec((tb, D), lambda i: (i, 0)),
        ],
        out_specs=[
            pl.BlockSpec((tb, feat_dim), lambda i: (i, 0)),
            pl.BlockSpec((tb, feat_dim), lambda i: (i, 0)),
            pl.BlockSpec((tb, feat_dim), lambda i: (i, 0)),
            pl.BlockSpec((tb, feat_dim), lambda i: (i, 0)),
        ],
    )
    pipe(x_hbm, x2_hbm, o1a, o2a, o1b, o2b)


@jax.jit
def _run(x, x2, w_head, b_head, w_proj, b_proj):
    B, D = x.shape
    F2 = w_proj.shape[1]
    feat_dim = F2 // 2
    tb = _pick_tile(B)
    steps = B // tb
    any_spec = pl.BlockSpec(memory_space=pltpu.MemorySpace.HBM)
    vmem_spec = pl.BlockSpec(memory_space=pltpu.MemorySpace.VMEM)
    return pl.pallas_call(
        functools.partial(_outer_body, feat_dim=feat_dim, tb=tb, steps=steps),
        out_shape=tuple(jax.ShapeDtypeStruct((B, feat_dim), x.dtype)
                        for _ in range(4)),
        in_specs=[any_spec, any_spec, vmem_spec, vmem_spec, vmem_spec, vmem_spec],
        out_specs=(any_spec, any_spec, any_spec, any_spec),
        scratch_shapes=[
            pltpu.VMEM((D, D), jnp.bfloat16),
            pltpu.VMEM((D, F2), jnp.bfloat16),
        ],
        compiler_params=pltpu.CompilerParams(
            vmem_limit_bytes=100 * 1024 * 1024,
        ),
    )(x, x2, w_head, b_head, w_proj, b_proj)


def kernel(x, x2, w_head, b_head, w_proj, b_proj):
    return _run(x, x2, w_head, b_head, w_proj, b_proj)
